# Initial kernel scaffold; baseline (speedup 1.0000x reference)
import jax
import jax.numpy as jnp
from jax.experimental import pallas as pl


def kernel(input_D_sum):
    raise NotImplementedError("write your pallas kernel here")


# speedup vs baseline: 1.5454x; 1.5454x over previous
import jax
import jax.numpy as jnp
from jax.experimental import pallas as pl
from jax.experimental.pallas import tpu as pltpu

_K = 16
_N = 256
_BIG = 1e9


def _ctable_kernel(d_ref, c_ref, call_ref, cprev_ref):
    kk = pl.program_id(1)
    d = d_ref[0]
    nn = jax.lax.broadcasted_iota(jnp.int32, (_N, _N), 0)
    ii = jax.lax.broadcasted_iota(jnp.int32, (_N, _N), 1)
    ik = jax.lax.broadcasted_iota(jnp.int32, (_N, _K), 1)

    @pl.when(kk == 0)
    def _init():
        col0 = d[:, _N - 1:_N]
        cprev_ref[:, :] = col0
        c_ref[0] = jnp.where(ik == 0, col0, 0.0)
        call_ref[0] = jnp.where(ii == _N - 1, 1.0, -1.0)

    @pl.when(kk > 0)
    def _step():
        cprev = cprev_ref[:, :]
        row_shift = jnp.sum(jnp.where(nn == ii + 1, cprev, 0.0), axis=0,
                            keepdims=True)
        a = d + row_shift
        valid = (ii >= nn) & (ii < _N - kk)
        a_safe = jnp.where(valid, a, _BIG)
        cmin = jnp.min(a_safe, axis=1, keepdims=True)
        nn_col = jax.lax.broadcasted_iota(jnp.int32, (_N, 1), 0)
        rowvalid = nn_col < _N - kk
        newcol = jnp.where(rowvalid, cmin, 0.0)
        cprev_ref[:, :] = newcol
        c_ref[0] = jnp.where(ik == kk, newcol, c_ref[0])
        w = jnp.where(valid, jnp.exp(cmin - a_safe), 0.0)
        s = jnp.sum(w, axis=1, keepdims=True)
        s = jnp.where(s > 0.0, s, 1.0)
        call_ref[0] = jnp.where(valid & rowvalid, w / s, -1.0)


def kernel(input_D_sum):
    b = input_D_sum.shape[0]
    c, call_flat = pl.pallas_call(
        _ctable_kernel,
        grid=(b, _K),
        in_specs=[pl.BlockSpec((1, _N, _N), lambda bi, kk: (bi, 0, 0))],
        out_specs=[
            pl.BlockSpec((1, _N, _K), lambda bi, kk: (bi, 0, 0)),
            pl.BlockSpec((1, _N, _N), lambda bi, kk: (bi, 0, kk)),
        ],
        out_shape=[
            jax.ShapeDtypeStruct((b, _N, _K), jnp.float32),
            jax.ShapeDtypeStruct((b, _N, _K * _N), jnp.float32),
        ],
        scratch_shapes=[pltpu.VMEM((_N, 1), jnp.float32)],
    )(input_D_sum)
    return c, call_flat.reshape(b, _N, _K, _N)
